# Initial kernel scaffold; baseline (speedup 1.0000x reference)
import functools

import jax
import jax.numpy as jnp
from jax import lax
from jax.experimental import pallas as pl
from jax.experimental.pallas import tpu as pltpu
from jax.experimental.pallas import tpu_sc as plsc

B, S, D = 8, 2048, 384
MAXM = 14336
NTILES = 32
TPB = NTILES // B
ROWS_PER_TILE = MAXM // TPB
CHUNK = 128
NCHUNK = ROWS_PER_TILE // CHUNK
SV = S // 16
CV = CHUNK // 16
DV = D // 16


@functools.partial(
    pl.kernel,
    out_type=(
        jax.ShapeDtypeStruct((B * MAXM, D), jnp.float32),
        jax.ShapeDtypeStruct((16,), jnp.int32),
    ),
    mesh=plsc.VectorSubcoreMesh(core_axis_name="c", subcore_axis_name="s"),
    scratch_types=[
        pltpu.VMEM((S,), jnp.int32),
        pltpu.VMEM((S,), jnp.int32),
        pltpu.VMEM((CHUNK,), jnp.int32),
        pltpu.VMEM((CHUNK, D), jnp.float32),
        pltpu.VMEM((CHUNK, D), jnp.float32),
        pltpu.VMEM((16,), jnp.int32),
        pltpu.SemaphoreType.DMA,
    ],
)
def _expand(x_hbm, dur_hbm, out_hbm, len_hbm,
            dur_ref, csum_ref, idx_ref, buf, zbuf, lens_v, sem):
    cid = lax.axis_index("c")
    sid = lax.axis_index("s")
    wid = cid * 16 + sid
    b = wid // TPB
    q = wid % TPB
    t0 = q * ROWS_PER_TILE
    out_base = b * MAXM + t0
    src_base = b * S

    iota16 = lax.broadcasted_iota(jnp.int32, (16,), 0)
    zv = jnp.zeros((16,), jnp.float32)

    pltpu.async_copy(dur_hbm.at[b], dur_ref, sem).wait()

    def cs_body(i, carry):
        v = dur_ref[pl.ds(i * 16, 16)]
        csum_ref[pl.ds(i * 16, 16)] = jnp.cumsum(v) + carry
        return carry + jnp.sum(v)

    total = lax.fori_loop(0, SV, cs_body, jnp.int32(0))

    def z_body(r, _):
        for col in range(DV):
            zbuf[r, pl.ds(col * 16, 16)] = zv
        return 0

    lax.fori_loop(0, CHUNK, z_body, 0)

    @pl.when(wid == 0)
    def _():
        def lens_body(bb, acc):
            pltpu.async_copy(dur_hbm.at[bb], dur_ref, sem).wait()

            def s_body(i, a):
                return a + dur_ref[pl.ds(i * 16, 16)]

            part = lax.fori_loop(0, SV, s_body, jnp.zeros((16,), jnp.int32))
            tt = jnp.sum(part)
            return jnp.where(iota16 == bb, tt, acc)

        lens_v[...] = lax.fori_loop(0, B, lens_body,
                                    jnp.zeros((16,), jnp.int32))
        pltpu.async_copy(lens_v, len_hbm, sem).wait()

    for c in range(NCHUNK):
        tc0 = t0 + c * CHUNK
        cut = jnp.clip(total - tc0, 0, CHUNK)
        dst = out_hbm.at[pl.ds(out_base + c * CHUNK, CHUNK)]

        @pl.when(cut > 0)
        def _(tc0=tc0, cut=cut, dst=dst):
            def idx_body(v, _):
                t = tc0 + v * 16 + iota16
                j = jnp.zeros((16,), jnp.int32)
                step = 1024
                for _u in range(11):
                    probe = plsc.load_gather(csum_ref, [j + (step - 1)])
                    j = jnp.where(probe <= t, j + step, j)
                    step //= 2
                idx_ref[pl.ds(v * 16, 16)] = src_base + jnp.minimum(j, S - 1)
                return 0

            lax.fori_loop(0, CV, idx_body, 0)
            pltpu.async_copy(x_hbm.at[idx_ref], buf, sem).wait()

            @pl.when(cut < CHUNK)
            def _():
                def zr(r, _):
                    for col in range(DV):
                        buf[r, pl.ds(col * 16, 16)] = zv
                    return 0

                lax.fori_loop(cut, CHUNK, zr, 0)

            pltpu.async_copy(buf, dst, sem).wait()

        @pl.when(cut <= 0)
        def _(dst=dst):
            pltpu.async_copy(zbuf, dst, sem).wait()


def kernel(x, duration, max_mel_len):
    del max_mel_len
    out_flat, lens16 = _expand(x.reshape(B * S, D), duration)
    return out_flat.reshape(B, MAXM, D), lens16[:B]


# speedup vs baseline: 67.6050x; 67.6050x over previous
import functools

import jax
import jax.numpy as jnp
from jax import lax
from jax.experimental import pallas as pl
from jax.experimental.pallas import tpu as pltpu
from jax.experimental.pallas import tpu_sc as plsc

B, S, D = 8, 2048, 384
MAXM = 14336
NTILES = 32
TPB = NTILES // B
ROWS_PER_TILE = MAXM // TPB
CHUNK = 128
NCHUNK = ROWS_PER_TILE // CHUNK
SV = S // 16
CV = CHUNK // 16
DV = D // 16


@functools.partial(
    pl.kernel,
    out_type=(
        jax.ShapeDtypeStruct((B * MAXM, D), jnp.float32),
        jax.ShapeDtypeStruct((16,), jnp.int32),
    ),
    mesh=plsc.VectorSubcoreMesh(core_axis_name="c", subcore_axis_name="s"),
    compiler_params=pltpu.CompilerParams(needs_layout_passes=False),
    scratch_types=[
        pltpu.VMEM((S,), jnp.int32),
        pltpu.VMEM((S,), jnp.int32),
        pltpu.VMEM((CHUNK,), jnp.int32),
        pltpu.VMEM((CHUNK, D), jnp.float32),
        pltpu.VMEM((CHUNK, D), jnp.float32),
        pltpu.VMEM((16,), jnp.int32),
        pltpu.VMEM((16,), jnp.int32),
        pltpu.SemaphoreType.DMA,
    ],
)
def _expand(x_hbm, dur_hbm, out_hbm, len_hbm,
            dur_ref, csum_ref, idx_ref, buf, zbuf, lens_v, tmp16, sem):
    cid = lax.axis_index("c")
    sid = lax.axis_index("s")
    wid = cid * 16 + sid
    b = wid // TPB
    q = wid % TPB
    t0 = q * ROWS_PER_TILE
    out_base = b * MAXM + t0
    src_base = b * S

    iota16 = lax.broadcasted_iota(jnp.int32, (16,), 0)
    zv = jnp.zeros((16,), jnp.float32)

    gather_dnums = lax.GatherDimensionNumbers(
        offset_dims=(), collapsed_slice_dims=(0,), start_index_map=(0,))

    def lane_permute(v, idx):
        return lax.gather(v, idx[:, None], gather_dnums, slice_sizes=(1,),
                          mode=lax.GatherScatterMode.PROMISE_IN_BOUNDS)

    def lane_cumsum(v):
        s = v
        for k in (1, 2, 4, 8):
            sh = lane_permute(s, jnp.maximum(iota16 - k, 0))
            s = s + jnp.where(iota16 >= k, sh, 0)
        return s

    pltpu.async_copy(dur_hbm.at[b], dur_ref, sem).wait()

    def cs_body(i, carry):
        v = dur_ref[pl.ds(i * 16, 16)]
        s = lane_cumsum(v) + carry
        csum_ref[pl.ds(i * 16, 16)] = s
        return s[15]

    total = lax.fori_loop(0, SV, cs_body, jnp.int32(0))

    def z_body(r, _):
        for col in range(DV):
            zbuf[r, pl.ds(col * 16, 16)] = zv
        return 0

    lax.fori_loop(0, CHUNK, z_body, 0)

    @pl.when(wid == 0)
    def _():
        def lens_body(bb, acc):
            pltpu.async_copy(dur_hbm.at[bb], dur_ref, sem).wait()

            def s_body(i, a):
                return a + dur_ref[pl.ds(i * 16, 16)]

            part = lax.fori_loop(0, SV, s_body, jnp.zeros((16,), jnp.int32))
            tt = lane_cumsum(part)[15]
            return jnp.where(iota16 == bb, tt, acc)

        lens_v[...] = lax.fori_loop(0, B, lens_body,
                                    jnp.zeros((16,), jnp.int32))
        pltpu.async_copy(lens_v, len_hbm, sem).wait()

    for c in range(NCHUNK):
        tc0 = t0 + c * CHUNK
        cut = jnp.clip(total - tc0, 0, CHUNK)
        dst = out_hbm.at[pl.ds(out_base + c * CHUNK, CHUNK)]

        @pl.when(cut > 0)
        def _(tc0=tc0, cut=cut, dst=dst):
            def idx_body(v, _):
                t = tc0 + v * 16 + iota16
                j = jnp.zeros((16,), jnp.int32)
                step = 1024
                for _u in range(11):
                    probe = plsc.load_gather(csum_ref, [j + (step - 1)])
                    j = jnp.where(probe <= t, j + step, j)
                    step //= 2
                idx_ref[pl.ds(v * 16, 16)] = src_base + jnp.minimum(j, S - 1)
                return 0

            lax.fori_loop(0, CV, idx_body, 0)
            pltpu.async_copy(x_hbm.at[idx_ref], buf, sem).wait()

            @pl.when(cut < CHUNK)
            def _():
                def zr(r, _):
                    for col in range(DV):
                        buf[r, pl.ds(col * 16, 16)] = zv
                    return 0

                lax.fori_loop(cut, CHUNK, zr, 0)

            pltpu.async_copy(buf, dst, sem).wait()

        @pl.when(cut <= 0)
        def _(dst=dst):
            pltpu.async_copy(zbuf, dst, sem).wait()


def kernel(x, duration, max_mel_len):
    del max_mel_len
    out_flat, lens16 = _expand(x.reshape(B * S, D), duration)
    return out_flat.reshape(B, MAXM, D), lens16[:B]


# speedup vs baseline: 74.1613x; 1.0970x over previous
import functools

import jax
import jax.numpy as jnp
from jax import lax
from jax.experimental import pallas as pl
from jax.experimental.pallas import tpu as pltpu
from jax.experimental.pallas import tpu_sc as plsc

B, S, D = 8, 2048, 384
MAXM = 14336
NTILES = 32
TPB = NTILES // B
ROWS_PER_TILE = MAXM // TPB
CHUNK = 64
NCHUNK = ROWS_PER_TILE // CHUNK
SV = S // 16
CV = CHUNK // 16
DV = D // 16


@functools.partial(
    pl.kernel,
    out_type=(
        jax.ShapeDtypeStruct((B * MAXM, D), jnp.float32),
        jax.ShapeDtypeStruct((16,), jnp.int32),
    ),
    mesh=plsc.VectorSubcoreMesh(core_axis_name="c", subcore_axis_name="s"),
    compiler_params=pltpu.CompilerParams(needs_layout_passes=False),
    scratch_types=[
        pltpu.VMEM((S,), jnp.int32),
        pltpu.VMEM((S,), jnp.int32),
        pltpu.VMEM((2, CHUNK), jnp.int32),
        pltpu.VMEM((CHUNK, D), jnp.float32),
        pltpu.VMEM((CHUNK, D), jnp.float32),
        pltpu.VMEM((CHUNK, D), jnp.float32),
        pltpu.VMEM((16,), jnp.int32),
        pltpu.SemaphoreType.DMA,
        pltpu.SemaphoreType.DMA,
        pltpu.SemaphoreType.DMA,
        pltpu.SemaphoreType.DMA,
        pltpu.SemaphoreType.DMA,
    ],
)
def _expand(x_hbm, dur_hbm, out_hbm, len_hbm,
            dur_ref, csum_ref, idx2, buf0, buf1, zbuf, lens_v,
            sem, gsem0, gsem1, ssem0, ssem1):
    cid = lax.axis_index("c")
    sid = lax.axis_index("s")
    wid = cid * 16 + sid
    b = wid // TPB
    q = wid % TPB
    t0 = q * ROWS_PER_TILE
    out_base = b * MAXM + t0
    src_base = b * S

    bufs = (buf0, buf1)
    gsems = (gsem0, gsem1)
    ssems = (ssem0, ssem1)

    iota16 = lax.broadcasted_iota(jnp.int32, (16,), 0)
    zv = jnp.zeros((16,), jnp.float32)

    gather_dnums = lax.GatherDimensionNumbers(
        offset_dims=(), collapsed_slice_dims=(0,), start_index_map=(0,))

    def lane_permute(v, idx):
        return lax.gather(v, idx[:, None], gather_dnums, slice_sizes=(1,),
                          mode=lax.GatherScatterMode.PROMISE_IN_BOUNDS)

    def lane_cumsum(v):
        s = v
        for k in (1, 2, 4, 8):
            sh = lane_permute(s, jnp.maximum(iota16 - k, 0))
            s = s + jnp.where(iota16 >= k, sh, 0)
        return s

    pltpu.async_copy(dur_hbm.at[b], dur_ref, sem).wait()

    def cs_body(i, carry):
        v = dur_ref[pl.ds(i * 16, 16)]
        s = lane_cumsum(v) + carry
        csum_ref[pl.ds(i * 16, 16)] = s
        return s[15]

    total = lax.fori_loop(0, SV, cs_body, jnp.int32(0))

    def z_body(r, _):
        for col in range(DV):
            zbuf[r, pl.ds(col * 16, 16)] = zv
        return 0

    lax.fori_loop(0, CHUNK, z_body, 0)

    @pl.when(wid == 0)
    def _():
        def lens_body(bb, acc):
            pltpu.async_copy(dur_hbm.at[bb], dur_ref, sem).wait()

            def s_body(i, a):
                return a + dur_ref[pl.ds(i * 16, 16)]

            part = lax.fori_loop(0, SV, s_body, jnp.zeros((16,), jnp.int32))
            tt = lane_cumsum(part)[15]
            return jnp.where(iota16 == bb, tt, acc)

        lens_v[...] = lax.fori_loop(0, B, lens_body,
                                    jnp.zeros((16,), jnp.int32))
        pltpu.async_copy(lens_v, len_hbm, sem).wait()

    def chunk_pair(g, _):
        for p in range(2):
            buf = bufs[p]
            cc = g * 2 + p
            tc0 = t0 + cc * CHUNK
            cut = jnp.clip(total - tc0, 0, CHUNK)
            dst = out_hbm.at[pl.ds(out_base + cc * CHUNK, CHUNK)]

            @pl.when(g > 0)
            def _(buf=buf, dst=dst, p=p):
                pltpu.make_async_copy(buf, dst, ssems[p]).wait()

            @pl.when(cut > 0)
            def _(p=p, buf=buf, tc0=tc0, cut=cut, dst=dst):
                def idx_body(v, _):
                    t = tc0 + v * 16 + iota16
                    j = jnp.zeros((16,), jnp.int32)
                    step = 1024
                    for _u in range(11):
                        probe = plsc.load_gather(csum_ref, [j + (step - 1)])
                        j = jnp.where(probe <= t, j + step, j)
                        step //= 2
                    idx2[p, pl.ds(v * 16, 16)] = (
                        src_base + jnp.minimum(j, S - 1))
                    return 0

                lax.fori_loop(0, CV, idx_body, 0)
                pltpu.async_copy(x_hbm.at[idx2.at[p]], buf, gsems[p]).wait()

                @pl.when(cut < CHUNK)
                def _():
                    def zr(r, _):
                        for col in range(DV):
                            buf[r, pl.ds(col * 16, 16)] = zv
                        return 0

                    lax.fori_loop(cut, CHUNK, zr, 0)

                pltpu.async_copy(buf, dst, ssems[p])

            @pl.when(cut <= 0)
            def _(p=p, dst=dst):
                pltpu.async_copy(zbuf, dst, ssems[p])

        return 0

    lax.fori_loop(0, NCHUNK // 2, chunk_pair, 0)

    for c in (NCHUNK - 2, NCHUNK - 1):
        p = c % 2
        dst = out_hbm.at[pl.ds(out_base + c * CHUNK, CHUNK)]
        pltpu.make_async_copy(bufs[p], dst, ssems[p]).wait()


def kernel(x, duration, max_mel_len):
    del max_mel_len
    out_flat, lens16 = _expand(x.reshape(B * S, D), duration)
    return out_flat.reshape(B, MAXM, D), lens16[:B]


# speedup vs baseline: 89.8618x; 1.2117x over previous
import functools

import jax
import jax.numpy as jnp
from jax import lax
from jax.experimental import pallas as pl
from jax.experimental.pallas import tpu as pltpu
from jax.experimental.pallas import tpu_sc as plsc

B, S, D = 8, 2048, 384
MAXM = 14336
NTILES = 32
TPB = NTILES // B
ROWS_PER_TILE = MAXM // TPB
CHUNK = 64
NCHUNK = ROWS_PER_TILE // CHUNK
SV = S // 16
CV = CHUNK // 16
DV = D // 16


@functools.partial(
    pl.kernel,
    out_type=(
        jax.ShapeDtypeStruct((B * MAXM, D), jnp.float32),
        jax.ShapeDtypeStruct((16,), jnp.int32),
    ),
    mesh=plsc.VectorSubcoreMesh(core_axis_name="c", subcore_axis_name="s"),
    compiler_params=pltpu.CompilerParams(needs_layout_passes=False),
    scratch_types=[
        pltpu.VMEM((S,), jnp.int32),
        pltpu.VMEM((S,), jnp.int32),
        pltpu.VMEM((2, CHUNK), jnp.int32),
        pltpu.VMEM((CHUNK, D), jnp.float32),
        pltpu.VMEM((CHUNK, D), jnp.float32),
        pltpu.VMEM((CHUNK, D), jnp.float32),
        pltpu.VMEM((16,), jnp.int32),
        pltpu.SemaphoreType.DMA,
        pltpu.SemaphoreType.DMA,
        pltpu.SemaphoreType.DMA,
        pltpu.SemaphoreType.DMA,
        pltpu.SemaphoreType.DMA,
    ],
)
def _expand(x_hbm, dur_hbm, out_hbm, len_hbm,
            dur_ref, csum_ref, idx2, buf0, buf1, zbuf, lens_v,
            sem, gsem0, gsem1, ssem0, ssem1):
    cid = lax.axis_index("c")
    sid = lax.axis_index("s")
    wid = cid * 16 + sid
    b = wid // TPB
    q = wid % TPB
    t0 = q * ROWS_PER_TILE
    out_base = b * MAXM + t0
    src_base = b * S

    bufs = (buf0, buf1)
    gsems = (gsem0, gsem1)
    ssems = (ssem0, ssem1)

    iota16 = lax.broadcasted_iota(jnp.int32, (16,), 0)
    zv = jnp.zeros((16,), jnp.float32)

    gather_dnums = lax.GatherDimensionNumbers(
        offset_dims=(), collapsed_slice_dims=(0,), start_index_map=(0,))

    def lane_permute(v, idx):
        return lax.gather(v, idx[:, None], gather_dnums, slice_sizes=(1,),
                          mode=lax.GatherScatterMode.PROMISE_IN_BOUNDS)

    def lane_cumsum(v):
        s = v
        for k in (1, 2, 4, 8):
            sh = lane_permute(s, jnp.maximum(iota16 - k, 0))
            s = s + jnp.where(iota16 >= k, sh, 0)
        return s

    pltpu.async_copy(dur_hbm.at[b], dur_ref, sem).wait()

    def cs_body(i, carry):
        v = dur_ref[pl.ds(i * 16, 16)]
        s = lane_cumsum(v) + carry
        csum_ref[pl.ds(i * 16, 16)] = s
        return s[15]

    total = lax.fori_loop(0, SV, cs_body, jnp.int32(0))

    def z_body(r, _):
        for col in range(DV):
            zbuf[r, pl.ds(col * 16, 16)] = zv
        return 0

    lax.fori_loop(0, CHUNK, z_body, 0)

    @pl.when(wid == 0)
    def _():
        def lens_body(bb, acc):
            pltpu.async_copy(dur_hbm.at[bb], dur_ref, sem).wait()

            def s_body(i, a):
                return a + dur_ref[pl.ds(i * 16, 16)]

            part = lax.fori_loop(0, SV, s_body, jnp.zeros((16,), jnp.int32))
            tt = lane_cumsum(part)[15]
            return jnp.where(iota16 == bb, tt, acc)

        lens_v[...] = lax.fori_loop(0, B, lens_body,
                                    jnp.zeros((16,), jnp.int32))
        pltpu.async_copy(lens_v, len_hbm, sem).wait()

    def chunk_pair(g, _):
        for p in range(2):
            buf = bufs[p]
            cbat = q + (g * 2 + p) * TPB
            tc0 = cbat * CHUNK
            cut = jnp.clip(total - tc0, 0, CHUNK)
            dst = out_hbm.at[pl.ds(b * MAXM + tc0, CHUNK)]

            @pl.when(g > 0)
            def _(buf=buf, dst=dst, p=p):
                pltpu.make_async_copy(buf, dst, ssems[p]).wait()

            @pl.when(cut > 0)
            def _(p=p, buf=buf, tc0=tc0, cut=cut, dst=dst):
                def idx_body(v, _):
                    t = tc0 + v * 16 + iota16
                    j = jnp.zeros((16,), jnp.int32)
                    step = 1024
                    for _u in range(11):
                        probe = plsc.load_gather(csum_ref, [j + (step - 1)])
                        j = jnp.where(probe <= t, j + step, j)
                        step //= 2
                    idx2[p, pl.ds(v * 16, 16)] = (
                        src_base + jnp.minimum(j, S - 1))
                    return 0

                lax.fori_loop(0, CV, idx_body, 0)
                pltpu.async_copy(x_hbm.at[idx2.at[p]], buf, gsems[p]).wait()

                @pl.when(cut < CHUNK)
                def _():
                    def zr(r, _):
                        for col in range(DV):
                            buf[r, pl.ds(col * 16, 16)] = zv
                        return 0

                    lax.fori_loop(cut, CHUNK, zr, 0)

                pltpu.async_copy(buf, dst, ssems[p])

            @pl.when(cut <= 0)
            def _(p=p, dst=dst):
                pltpu.async_copy(zbuf, dst, ssems[p])

        return 0

    lax.fori_loop(0, NCHUNK // 2, chunk_pair, 0)

    for c in (NCHUNK - 2, NCHUNK - 1):
        p = c % 2
        dst = out_hbm.at[pl.ds(out_base + c * CHUNK, CHUNK)]
        pltpu.make_async_copy(bufs[p], dst, ssems[p]).wait()


def kernel(x, duration, max_mel_len):
    del max_mel_len
    out_flat, lens16 = _expand(x.reshape(B * S, D), duration)
    return out_flat.reshape(B, MAXM, D), lens16[:B]


# speedup vs baseline: 93.8874x; 1.0448x over previous
import functools

import jax
import jax.numpy as jnp
from jax import lax
from jax.experimental import pallas as pl
from jax.experimental.pallas import tpu as pltpu
from jax.experimental.pallas import tpu_sc as plsc

B, S, D = 8, 2048, 384
MAXM = 14336
NTILES = 32
TPB = NTILES // B
ROWS_PER_TILE = MAXM // TPB
CHUNK = 128
NCHUNK = ROWS_PER_TILE // CHUNK
SV = S // 16
CV = CHUNK // 16
DV = D // 16


@functools.partial(
    pl.kernel,
    out_type=(
        jax.ShapeDtypeStruct((B * MAXM, D), jnp.float32),
        jax.ShapeDtypeStruct((16,), jnp.int32),
    ),
    mesh=plsc.VectorSubcoreMesh(core_axis_name="c", subcore_axis_name="s"),
    compiler_params=pltpu.CompilerParams(needs_layout_passes=False),
    scratch_types=[
        pltpu.VMEM((S,), jnp.int32),
        pltpu.VMEM((S,), jnp.int32),
        pltpu.VMEM((2, CHUNK), jnp.int32),
        pltpu.VMEM((CHUNK, D), jnp.float32),
        pltpu.VMEM((CHUNK, D), jnp.float32),
        pltpu.VMEM((CHUNK // 2, D), jnp.float32),
        pltpu.VMEM((16,), jnp.int32),
        pltpu.SemaphoreType.DMA,
        pltpu.SemaphoreType.DMA,
        pltpu.SemaphoreType.DMA,
        pltpu.SemaphoreType.DMA,
        pltpu.SemaphoreType.DMA,
    ],
)
def _expand(x_hbm, dur_hbm, out_hbm, len_hbm,
            dur_ref, csum_ref, idx2, buf0, buf1, zbuf, lens_v,
            sem, gsem0, gsem1, ssem0, ssem1):
    cid = lax.axis_index("c")
    sid = lax.axis_index("s")
    wid = cid * 16 + sid
    b = wid // TPB
    q = wid % TPB
    t0 = q * ROWS_PER_TILE
    out_base = b * MAXM + t0
    src_base = b * S

    bufs = (buf0, buf1)
    gsems = (gsem0, gsem1)
    ssems = (ssem0, ssem1)

    iota16 = lax.broadcasted_iota(jnp.int32, (16,), 0)
    zv = jnp.zeros((16,), jnp.float32)

    gather_dnums = lax.GatherDimensionNumbers(
        offset_dims=(), collapsed_slice_dims=(0,), start_index_map=(0,))

    def lane_permute(v, idx):
        return lax.gather(v, idx[:, None], gather_dnums, slice_sizes=(1,),
                          mode=lax.GatherScatterMode.PROMISE_IN_BOUNDS)

    def lane_cumsum(v):
        s = v
        for k in (1, 2, 4, 8):
            sh = lane_permute(s, jnp.maximum(iota16 - k, 0))
            s = s + jnp.where(iota16 >= k, sh, 0)
        return s

    pltpu.async_copy(dur_hbm.at[b], dur_ref, sem).wait()

    def cs_body(i, carry):
        v = dur_ref[pl.ds(i * 16, 16)]
        s = lane_cumsum(v) + carry
        csum_ref[pl.ds(i * 16, 16)] = s
        return s[15]

    total = lax.fori_loop(0, SV, cs_body, jnp.int32(0))

    def z_body(r, _):
        for col in range(DV):
            zbuf[r, pl.ds(col * 16, 16)] = zv
        return 0

    lax.fori_loop(0, CHUNK // 2, z_body, 0)

    @pl.when(wid == 0)
    def _():
        def lens_body(bb, acc):
            pltpu.async_copy(dur_hbm.at[bb], dur_ref, sem).wait()

            def s_body(i, a):
                return a + dur_ref[pl.ds(i * 16, 16)]

            part = lax.fori_loop(0, SV, s_body, jnp.zeros((16,), jnp.int32))
            tt = lane_cumsum(part)[15]
            return jnp.where(iota16 == bb, tt, acc)

        lens_v[...] = lax.fori_loop(0, B, lens_body,
                                    jnp.zeros((16,), jnp.int32))
        pltpu.async_copy(lens_v, len_hbm, sem).wait()

    def chunk_pair(g, _):
        for p in range(2):
            buf = bufs[p]
            cbat = q + (g * 2 + p) * TPB
            tc0 = cbat * CHUNK
            cut = jnp.clip(total - tc0, 0, CHUNK)
            dst = out_hbm.at[pl.ds(b * MAXM + tc0, CHUNK)]

            @pl.when(g > 0)
            def _(buf=buf, dst=dst, p=p):
                pltpu.make_async_copy(buf, dst, ssems[p]).wait()

            @pl.when(cut > 0)
            def _(p=p, buf=buf, tc0=tc0, cut=cut, dst=dst):
                def idx_body(v, _):
                    t = tc0 + v * 16 + iota16
                    j = jnp.zeros((16,), jnp.int32)
                    step = 1024
                    for _u in range(11):
                        probe = plsc.load_gather(csum_ref, [j + (step - 1)])
                        j = jnp.where(probe <= t, j + step, j)
                        step //= 2
                    idx2[p, pl.ds(v * 16, 16)] = (
                        src_base + jnp.minimum(j, S - 1))
                    return 0

                lax.fori_loop(0, CV, idx_body, 0)
                pltpu.async_copy(x_hbm.at[idx2.at[p]], buf, gsems[p]).wait()

                @pl.when(cut < CHUNK)
                def _():
                    def zr(r, _):
                        for col in range(DV):
                            buf[r, pl.ds(col * 16, 16)] = zv
                        return 0

                    lax.fori_loop(cut, CHUNK, zr, 0)

                pltpu.async_copy(buf, dst, ssems[p])

            @pl.when(cut <= 0)
            def _(p=p, tc0=tc0):
                lo = out_hbm.at[pl.ds(b * MAXM + tc0, CHUNK // 2)]
                hi = out_hbm.at[pl.ds(b * MAXM + tc0 + CHUNK // 2,
                                      CHUNK // 2)]
                pltpu.async_copy(zbuf, lo, ssems[p])
                pltpu.async_copy(zbuf, hi, ssems[p])

        return 0

    lax.fori_loop(0, NCHUNK // 2, chunk_pair, 0)

    for c in (NCHUNK - 2, NCHUNK - 1):
        p = c % 2
        dst = out_hbm.at[pl.ds(out_base + c * CHUNK, CHUNK)]
        pltpu.make_async_copy(bufs[p], dst, ssems[p]).wait()


def kernel(x, duration, max_mel_len):
    del max_mel_len
    out_flat, lens16 = _expand(x.reshape(B * S, D), duration)
    return out_flat.reshape(B, MAXM, D), lens16[:B]
